# baseline (speedup 1.0000x reference)
import functools

import jax
import jax.numpy as jnp
from jax import lax
from jax.experimental import pallas as pl
from jax.experimental.pallas import tpu as pltpu
from jax.experimental.pallas import tpu_sc as plsc

B = 16384
EMB = 32
GPAD = 16
BLK = 2048
GRID = B // BLK
CHUNK = 16


@functools.cache
def _build_gather():
    info = plsc.get_sparse_core_info()
    nc, ns = info.num_cores, info.num_subcores
    nw = nc * ns
    b_per_w = B // nw
    n_chunks = b_per_w // CHUNK
    mesh = plsc.VectorSubcoreMesh(core_axis_name="c", subcore_axis_name="s")

    @functools.partial(
        pl.kernel,
        mesh=mesh,
        out_type=jax.ShapeDtypeStruct((EMB, B), jnp.float32),
        scratch_types=[
            pltpu.VMEM((b_per_w,), jnp.int32),
            pltpu.VMEM((EMB, b_per_w), jnp.float32),
            pltpu.VMEM((EMB, CHUNK * 8), jnp.float32),
            pltpu.VMEM((EMB, CHUNK * 8), jnp.float32),
            pltpu.SemaphoreType.DMA,
            pltpu.SemaphoreType.DMA,
        ],
        compiler_params=pltpu.CompilerParams(
            use_tc_tiling_on_sc=False, needs_layout_passes=False),
    )
    def gather(table_t_hbm, idx_hbm, out_hbm, idx_v, cols_v,
               stage_a, stage_b, sem_a, sem_b):
        wid = lax.axis_index("s") * nc + lax.axis_index("c")
        base = wid * b_per_w
        iota = lax.iota(jnp.int32, CHUNK)

        pltpu.sync_copy(idx_hbm.at[pl.ds(base, b_per_w)], idx_v)
        sems = {id(stage_a): sem_a, id(stage_b): sem_b}

        def chunk_idx(c):
            return idx_v[pl.ds(c * CHUNK, CHUNK)]

        def fire(c, stage):
            v = chunk_idx(c)
            for k in range(CHUNK):
                r8 = pl.multiple_of((v[k] >> 3) << 3, 8)
                pltpu.async_copy(
                    table_t_hbm.at[:, pl.ds(r8, 8)],
                    stage.at[:, pl.ds(k * 8, 8)], sems[id(stage)])

        def drain(c, stage):
            v = chunk_idx(c)
            for k in range(CHUNK):
                r8 = pl.multiple_of((v[k] >> 3) << 3, 8)
                pltpu.make_async_copy(
                    table_t_hbm.at[:, pl.ds(r8, 8)],
                    stage.at[:, pl.ds(k * 8, 8)], sems[id(stage)]).wait()

        def select(c, stage):
            v = chunk_idx(c)
            src_col = (v & 7) + iota * 8
            dst_col = c * CHUNK + iota
            for i in range(EMB):
                row = jnp.full((CHUNK,), i, jnp.int32)
                vals = plsc.load_gather(stage, [row, src_col])
                plsc.store_scatter(cols_v, [row, dst_col], vals)

        fire(0, stage_a)

        def body(t, _):
            c = 2 * t
            fire(c + 1, stage_b)
            drain(c, stage_a)
            select(c, stage_a)
            fire(c + 2, stage_a)
            drain(c + 1, stage_b)
            select(c + 1, stage_b)
            return ()

        lax.fori_loop(0, n_chunks // 2 - 1, body, (), unroll=False)
        fire(n_chunks - 1, stage_b)
        drain(n_chunks - 2, stage_a)
        select(n_chunks - 2, stage_a)
        drain(n_chunks - 1, stage_b)
        select(n_chunks - 1, stage_b)

        pltpu.sync_copy(cols_v, out_hbm.at[:, pl.ds(base, b_per_w)])

    return gather


def _mlp_body(cat_ref, ut_ref, gt_ref, w1_ref, b1_ref, w2_ref, b2_ref,
              w3_ref, b3_ref, out_ref):
    c0 = (((0,), (0,)), ((), ()))
    ut = ut_ref[...]
    cat = cat_ref[0, 0, :]
    row = lax.broadcasted_iota(jnp.int32, (GPAD, BLK), 0)
    onehot_t = (row == cat[None, :]).astype(jnp.float32)
    g_t = lax.dot_general(gt_ref[...], onehot_t, c0,
                          preferred_element_type=jnp.float32)
    x_t = jnp.concatenate([ut, g_t], axis=0)
    h = jnp.maximum(
        lax.dot_general(w1_ref[...], x_t, c0,
                        preferred_element_type=jnp.float32)
        + b1_ref[...], 0.0)
    h = jnp.maximum(
        lax.dot_general(w2_ref[...], h, c0,
                        preferred_element_type=jnp.float32)
        + b2_ref[...], 0.0)
    out_ref[...] = (
        lax.dot_general(w3_ref[...], h, c0,
                        preferred_element_type=jnp.float32)
        + b3_ref[...])


@functools.cache
def _build_mlp(interpret=False):
    full = lambda *shape: pl.BlockSpec(shape, lambda i: (0,) * len(shape))
    return pl.pallas_call(
        _mlp_body,
        grid=(GRID,),
        in_specs=[
            pl.BlockSpec((1, 1, BLK), lambda i: (i, 0, 0)),
            pl.BlockSpec((EMB, BLK), lambda i: (0, i)),
            full(GPAD, EMB),
            full(2 * EMB, 128), full(128, 1),
            full(128, 64), full(64, 1),
            full(64, EMB), full(EMB, 1),
        ],
        out_specs=pl.BlockSpec((EMB, BLK), lambda i: (0, i)),
        out_shape=jax.ShapeDtypeStruct((EMB, B), jnp.float32),
        interpret=interpret,
    )


def kernel(customer_id, category_by_Gender, user_table, gender_table,
           W1, b1, W2, b2, W3, b3):
    cid = customer_id.astype(jnp.int32)
    cat = category_by_Gender.astype(jnp.int32).reshape(GRID, 1, BLK)
    u_t = _build_gather()(user_table.T, cid)
    gt_pad = jnp.pad(gender_table, ((0, GPAD - gender_table.shape[0]), (0, 0)))
    out_t = _build_mlp()(
        cat, u_t, gt_pad,
        W1, b1.reshape(-1, 1), W2, b2.reshape(-1, 1), W3, b3.reshape(-1, 1))
    return out_t.T


# speedup vs baseline: 1.4516x; 1.4516x over previous
import functools

import jax
import jax.numpy as jnp
from jax import lax
from jax.experimental import pallas as pl
from jax.experimental.pallas import tpu as pltpu
from jax.experimental.pallas import tpu_sc as plsc

B = 16384
EMB = 32
ROWP = 128
GPAD = 16
BLK = 2048
GRID = B // BLK

FCOLS = 1664
FGRID = 61


def _fmt_body(in_ref, out_ref):
    x = in_ref[...]
    out_ref[...] = jnp.concatenate(
        [x.T, jnp.zeros((FCOLS, ROWP - EMB), jnp.float32)], axis=1)


@functools.cache
def _build_fmt(interpret=False):
    return pl.pallas_call(
        _fmt_body,
        grid=(FGRID,),
        in_specs=[pl.BlockSpec((EMB, FCOLS), lambda i: (0, i))],
        out_specs=pl.BlockSpec((FCOLS, ROWP), lambda i: (i, 0)),
        out_shape=jax.ShapeDtypeStruct((FGRID * FCOLS, ROWP), jnp.float32),
        interpret=interpret,
    )


@functools.cache
def _build_gather():
    info = plsc.get_sparse_core_info()
    nc, ns = info.num_cores, info.num_subcores
    nw = nc * ns
    b_per_w = B // nw
    mesh = plsc.VectorSubcoreMesh(core_axis_name="c", subcore_axis_name="s")

    @functools.partial(
        pl.kernel,
        mesh=mesh,
        out_type=jax.ShapeDtypeStruct((B, ROWP), jnp.float32),
        scratch_types=[
            pltpu.VMEM((b_per_w,), jnp.int32),
            pltpu.VMEM((b_per_w, ROWP), jnp.float32),
            pltpu.SemaphoreType.DMA,
        ],
        compiler_params=pltpu.CompilerParams(use_tc_tiling_on_sc=False),
    )
    def gather(table_hbm, idx_hbm, out_hbm, idx_v, rows_v, sem):
        wid = lax.axis_index("s") * nc + lax.axis_index("c")
        base = wid * b_per_w
        pltpu.sync_copy(idx_hbm.at[pl.ds(base, b_per_w)], idx_v)
        pltpu.async_copy(table_hbm.at[idx_v], rows_v, sem).wait()
        pltpu.sync_copy(rows_v, out_hbm.at[pl.ds(base, b_per_w)])

    return gather


def _mlp_body(cat_ref, u_ref, gt_ref, w1_ref, b1_ref, w2_ref, b2_ref,
              w3_ref, b3_ref, out_ref):
    u = u_ref[:, :EMB]
    cat = cat_ref[0, 0, :]
    col = lax.broadcasted_iota(jnp.int32, (BLK, GPAD), 1)
    onehot = (col == cat[:, None]).astype(jnp.float32)
    g = jnp.dot(onehot, gt_ref[...], preferred_element_type=jnp.float32)
    x = jnp.concatenate([u, g], axis=1)
    h = jnp.maximum(
        jnp.dot(x, w1_ref[...], preferred_element_type=jnp.float32)
        + b1_ref[...], 0.0)
    h = jnp.maximum(
        jnp.dot(h, w2_ref[...], preferred_element_type=jnp.float32)
        + b2_ref[...], 0.0)
    out = (jnp.dot(h, w3_ref[...], preferred_element_type=jnp.float32)
           + b3_ref[...])
    out_ref[...] = out.T


@functools.cache
def _build_mlp(interpret=False):
    full = lambda *shape: pl.BlockSpec(shape, lambda i: (0,) * len(shape))
    return pl.pallas_call(
        _mlp_body,
        grid=(GRID,),
        in_specs=[
            pl.BlockSpec((1, 1, BLK), lambda i: (i, 0, 0)),
            pl.BlockSpec((BLK, ROWP), lambda i: (i, 0)),
            full(GPAD, EMB),
            full(2 * EMB, 128), full(1, 128),
            full(128, 64), full(1, 64),
            full(64, EMB), full(1, EMB),
        ],
        out_specs=pl.BlockSpec((EMB, BLK), lambda i: (0, i)),
        out_shape=jax.ShapeDtypeStruct((EMB, B), jnp.float32),
        interpret=interpret,
    )


def kernel(customer_id, category_by_Gender, user_table, gender_table,
           W1, b1, W2, b2, W3, b3):
    cid = customer_id.astype(jnp.int32)
    cat = category_by_Gender.astype(jnp.int32).reshape(GRID, 1, BLK)
    table_p = _build_fmt()(user_table.T)
    user_emb = _build_gather()(table_p, cid)
    gt_pad = jnp.pad(gender_table, ((0, GPAD - gender_table.shape[0]), (0, 0)))
    out_t = _build_mlp()(
        cat, user_emb, gt_pad,
        W1, b1.reshape(1, -1), W2, b2.reshape(1, -1), W3, b3.reshape(1, -1))
    return out_t.T
